# baseline (speedup 1.0000x reference)
import functools

import jax
import jax.numpy as jnp
from jax import lax
from jax.experimental import pallas as pl
from jax.experimental.pallas import tpu as pltpu
from jax.experimental.pallas import tpu_sc as plsc

_NC = 2
_NS = 16
_NW = _NC * _NS
_CHUNK = 128
_TBLK = 2048


def _tc_transpose_body(a_ref, b_ref, ao_ref, bo_ref):
    ao_ref[...] = a_ref[...].T
    bo_ref[...] = b_ref[...].T


def _tc_transpose_half1(a, b, total_b):
    Bh, D = a.shape
    out = jax.ShapeDtypeStruct((D, total_b), jnp.float32)
    return pl.pallas_call(
        _tc_transpose_body,
        grid=(Bh // _TBLK,),
        in_specs=[pl.BlockSpec((_TBLK, D), lambda i: (i, 0)),
                  pl.BlockSpec((_TBLK, D), lambda i: (i, 0))],
        out_specs=[pl.BlockSpec((D, _TBLK), lambda i: (0, i)),
                   pl.BlockSpec((D, _TBLK), lambda i: (0, i))],
        out_shape=(out, out),
    )(a, b)


def _tc_transpose_body2(pm_ref, ps_ref, a_ref, b_ref, ao_ref, bo_ref):
    del pm_ref, ps_ref
    ao_ref[...] = a_ref[...].T
    bo_ref[...] = b_ref[...].T


def _tc_transpose_half2(prev_m, prev_s, a, b):
    Bh, D = a.shape
    total_b = prev_m.shape[1]
    off = (total_b - Bh) // _TBLK
    out = jax.ShapeDtypeStruct((D, total_b), jnp.float32)
    hbm = pl.BlockSpec(memory_space=pltpu.MemorySpace.HBM)
    return pl.pallas_call(
        _tc_transpose_body2,
        grid=(Bh // _TBLK,),
        in_specs=[hbm, hbm,
                  pl.BlockSpec((_TBLK, D), lambda i: (i, 0)),
                  pl.BlockSpec((_TBLK, D), lambda i: (i, 0))],
        out_specs=[pl.BlockSpec((D, _TBLK), lambda i, o=off: (0, o + i)),
                   pl.BlockSpec((D, _TBLK), lambda i, o=off: (0, o + i))],
        out_shape=(out, out),
        input_output_aliases={0: 0, 1: 1},
    )(prev_m, prev_s, a, b)


def _sc_rowgather_body(mus_hbm, sig_hbm, z_hbm, muz_hbm, sigz_hbm,
                       z_v, rows_v, gs0, gs1, ss0, ss1):
    N, D = mus_hbm.shape
    B = z_hbm.shape[0]
    b_per_w = B // _NW
    n_g = b_per_w // _CHUNK

    wid = lax.axis_index("s") * _NC + lax.axis_index("c")
    base = wid * b_per_w

    for g in range(n_g):
        pltpu.sync_copy(z_hbm.at[pl.ds(base + g * _CHUNK, _CHUNK)], z_v.at[g])

    gsems = (gs0, gs1)
    ssems = (ss0, ss1)
    items = [(src, dst, g)
             for src, dst in ((mus_hbm, muz_hbm), (sig_hbm, sigz_hbm))
             for g in range(n_g)]
    n = len(items)
    pend_g = [None, None]
    pend_s = [None, None]

    def issue_gather(i, b):
        src, _, g = items[i]
        pend_g[b] = pltpu.async_copy(src.at[z_v.at[g]], rows_v.at[b], gsems[b])

    issue_gather(0, 0)
    for i in range(n):
        b = i % 2
        if i + 1 < n:
            b2 = (i + 1) % 2
            if pend_s[b2] is not None:
                pend_s[b2].wait()
                pend_s[b2] = None
            issue_gather(i + 1, b2)
        pend_g[b].wait()
        _, dst, g = items[i]
        pend_s[b] = pltpu.async_copy(
            rows_v.at[b], dst.at[pl.ds(base + g * _CHUNK, _CHUNK)], ssems[b])
    for b in range(2):
        if pend_s[b] is not None:
            pend_s[b].wait()


def _sc_gather(mus_t, sig_t, z_half):
    N, D = mus_t.shape
    Bh = z_half.shape[0]
    out_t = jax.ShapeDtypeStruct((Bh, D), jnp.float32)
    mesh = plsc.VectorSubcoreMesh(core_axis_name="c", subcore_axis_name="s")
    b_per_w = Bh // _NW
    n_g = b_per_w // _CHUNK
    k = functools.partial(
        pl.kernel,
        out_type=(out_t, out_t),
        mesh=mesh,
        scratch_types=[
            pltpu.VMEM((n_g, _CHUNK), jnp.int32),
            pltpu.VMEM((2, _CHUNK, D), jnp.float32),
            pltpu.SemaphoreType.DMA,
            pltpu.SemaphoreType.DMA,
            pltpu.SemaphoreType.DMA,
            pltpu.SemaphoreType.DMA,
        ],
        compiler_params=pltpu.CompilerParams(needs_layout_passes=False),
    )(_sc_rowgather_body)
    return k(mus_t, sig_t, z_half)


def kernel(mus, sigmas, z):
    D, N = mus.shape
    B = z.shape[0]
    mus_t = mus.T
    sig_t = sigmas.T
    Bh = B // 2
    muz1, sigz1 = _sc_gather(mus_t, sig_t, z[:Bh])
    muz2, sigz2 = _sc_gather(mus_t, sig_t, z[Bh:])
    pm, ps = _tc_transpose_half1(muz1, sigz1, B)
    return _tc_transpose_half2(pm, ps, muz2, sigz2)


# speedup vs baseline: 1.0284x; 1.0284x over previous
import functools

import jax
import jax.numpy as jnp
from jax import lax
from jax.experimental import pallas as pl
from jax.experimental.pallas import tpu as pltpu
from jax.experimental.pallas import tpu_sc as plsc

_NC = 2
_NS = 16
_NW = _NC * _NS
_CHUNK = 128
_TBLK = 2048


def _tc_transpose_body(a_ref, b_ref, ao_ref, bo_ref):
    ao_ref[...] = a_ref[...].T
    bo_ref[...] = b_ref[...].T


def _tc_transpose_half1(a, b, total_b):
    Bh, D = a.shape
    out = jax.ShapeDtypeStruct((D, total_b), jnp.float32)
    return pl.pallas_call(
        _tc_transpose_body,
        grid=(Bh // _TBLK,),
        in_specs=[pl.BlockSpec((_TBLK, D), lambda i: (i, 0)),
                  pl.BlockSpec((_TBLK, D), lambda i: (i, 0))],
        out_specs=[pl.BlockSpec((D, _TBLK), lambda i: (0, i)),
                   pl.BlockSpec((D, _TBLK), lambda i: (0, i))],
        out_shape=(out, out),
    )(a, b)


def _tc_transpose_body2(pm_ref, ps_ref, a_ref, b_ref, ao_ref, bo_ref):
    del pm_ref, ps_ref
    ao_ref[...] = a_ref[...].T
    bo_ref[...] = b_ref[...].T


def _tc_transpose_half2(prev_m, prev_s, a, b):
    Bh, D = a.shape
    total_b = prev_m.shape[1]
    off = (total_b - Bh) // _TBLK
    out = jax.ShapeDtypeStruct((D, total_b), jnp.float32)
    hbm = pl.BlockSpec(memory_space=pltpu.MemorySpace.HBM)
    return pl.pallas_call(
        _tc_transpose_body2,
        grid=(Bh // _TBLK,),
        in_specs=[hbm, hbm,
                  pl.BlockSpec((_TBLK, D), lambda i: (i, 0)),
                  pl.BlockSpec((_TBLK, D), lambda i: (i, 0))],
        out_specs=[pl.BlockSpec((D, _TBLK), lambda i, o=off: (0, o + i)),
                   pl.BlockSpec((D, _TBLK), lambda i, o=off: (0, o + i))],
        out_shape=(out, out),
        input_output_aliases={0: 0, 1: 1},
    )(prev_m, prev_s, a, b)


def _sc_rowgather_body(mus_hbm, sig_hbm, z_hbm, muz_hbm, sigz_hbm,
                       z_v, rows_v, gs0, gs1, ss0, ss1, *, z_off, bh):
    N, D = mus_hbm.shape
    b_per_w = bh // _NW
    n_g = b_per_w // _CHUNK

    wid = lax.axis_index("s") * _NC + lax.axis_index("c")
    base = wid * b_per_w

    for g in range(n_g):
        pltpu.sync_copy(
            z_hbm.at[pl.ds(z_off + base + g * _CHUNK, _CHUNK)], z_v.at[g])

    gsems = (gs0, gs1)
    ssems = (ss0, ss1)
    items = [(src, dst, g)
             for src, dst in ((mus_hbm, muz_hbm), (sig_hbm, sigz_hbm))
             for g in range(n_g)]
    n = len(items)
    pend_g = [None, None]
    pend_s = [None, None]

    def issue_gather(i, b):
        src, _, g = items[i]
        pend_g[b] = pltpu.async_copy(src.at[z_v.at[g]], rows_v.at[b], gsems[b])

    issue_gather(0, 0)
    for i in range(n):
        b = i % 2
        if i + 1 < n:
            b2 = (i + 1) % 2
            if pend_s[b2] is not None:
                pend_s[b2].wait()
                pend_s[b2] = None
            issue_gather(i + 1, b2)
        pend_g[b].wait()
        _, dst, g = items[i]
        pend_s[b] = pltpu.async_copy(
            rows_v.at[b], dst.at[pl.ds(base + g * _CHUNK, _CHUNK)], ssems[b])
    for b in range(2):
        if pend_s[b] is not None:
            pend_s[b].wait()


def _sc_gather(mus_t, sig_t, z, z_off, bh):
    N, D = mus_t.shape
    out_t = jax.ShapeDtypeStruct((bh, D), jnp.float32)
    mesh = plsc.VectorSubcoreMesh(core_axis_name="c", subcore_axis_name="s")
    b_per_w = bh // _NW
    n_g = b_per_w // _CHUNK
    k = functools.partial(
        pl.kernel,
        out_type=(out_t, out_t),
        mesh=mesh,
        scratch_types=[
            pltpu.VMEM((n_g, _CHUNK), jnp.int32),
            pltpu.VMEM((2, _CHUNK, D), jnp.float32),
            pltpu.SemaphoreType.DMA,
            pltpu.SemaphoreType.DMA,
            pltpu.SemaphoreType.DMA,
            pltpu.SemaphoreType.DMA,
        ],
        compiler_params=pltpu.CompilerParams(needs_layout_passes=False),
    )(functools.partial(_sc_rowgather_body, z_off=z_off, bh=bh))
    return k(mus_t, sig_t, z)


def kernel(mus, sigmas, z):
    D, N = mus.shape
    B = z.shape[0]
    mus_t = mus.T
    sig_t = sigmas.T
    b1 = (3 * B // 4 // _TBLK) * _TBLK
    b2 = B - b1
    muz1, sigz1 = _sc_gather(mus_t, sig_t, z, 0, b1)
    muz2, sigz2 = _sc_gather(mus_t, sig_t, z, b1, b2)
    pm, ps = _tc_transpose_half1(muz1, sigz1, B)
    return _tc_transpose_half2(pm, ps, muz2, sigz2)


# speedup vs baseline: 1.0745x; 1.0448x over previous
import functools

import jax
import jax.numpy as jnp
from jax import lax
from jax.experimental import pallas as pl
from jax.experimental.pallas import tpu as pltpu
from jax.experimental.pallas import tpu_sc as plsc

_NC = 2
_NS = 16
_NW = _NC * _NS
_CHUNK = 128
_TBLK = 2048


def _tc_transpose_body(a_ref, b_ref, ao_ref, bo_ref):
    ao_ref[...] = a_ref[...].T
    bo_ref[...] = b_ref[...].T


def _tc_transpose_half1(a, b, total_b):
    Bh, D = a.shape
    out = jax.ShapeDtypeStruct((D, total_b), jnp.float32)
    return pl.pallas_call(
        _tc_transpose_body,
        grid=(Bh // _TBLK,),
        in_specs=[pl.BlockSpec((_TBLK, D), lambda i: (i, 0)),
                  pl.BlockSpec((_TBLK, D), lambda i: (i, 0))],
        out_specs=[pl.BlockSpec((D, _TBLK), lambda i: (0, i)),
                   pl.BlockSpec((D, _TBLK), lambda i: (0, i))],
        out_shape=(out, out),
    )(a, b)


def _tc_transpose_body2(pm_ref, ps_ref, a_ref, b_ref, ao_ref, bo_ref):
    del pm_ref, ps_ref
    ao_ref[...] = a_ref[...].T
    bo_ref[...] = b_ref[...].T


def _tc_transpose_half2(prev_m, prev_s, a, b):
    Bh, D = a.shape
    total_b = prev_m.shape[1]
    off = (total_b - Bh) // _TBLK
    out = jax.ShapeDtypeStruct((D, total_b), jnp.float32)
    hbm = pl.BlockSpec(memory_space=pltpu.MemorySpace.HBM)
    return pl.pallas_call(
        _tc_transpose_body2,
        grid=(Bh // _TBLK,),
        in_specs=[hbm, hbm,
                  pl.BlockSpec((_TBLK, D), lambda i: (i, 0)),
                  pl.BlockSpec((_TBLK, D), lambda i: (i, 0))],
        out_specs=[pl.BlockSpec((D, _TBLK), lambda i, o=off: (0, o + i)),
                   pl.BlockSpec((D, _TBLK), lambda i, o=off: (0, o + i))],
        out_shape=(out, out),
        input_output_aliases={0: 0, 1: 1},
    )(prev_m, prev_s, a, b)


def _sc_rowgather_body(mus_hbm, sig_hbm, z_hbm, muz_hbm, sigz_hbm,
                       z_v, rows_v, gs0, gs1, ss0, ss1, *, z_off, bh):
    N, D = mus_hbm.shape
    b_per_w = bh // _NW
    n_g = b_per_w // _CHUNK

    wid = lax.axis_index("s") * _NC + lax.axis_index("c")
    base = wid * b_per_w

    for g in range(n_g):
        pltpu.sync_copy(
            z_hbm.at[pl.ds(z_off + base + g * _CHUNK, _CHUNK)], z_v.at[g])

    gsems = (gs0, gs1)
    ssems = (ss0, ss1)
    items = [(src, dst, g)
             for src, dst in ((mus_hbm, muz_hbm), (sig_hbm, sigz_hbm))
             for g in range(n_g)]
    n = len(items)
    pend_g = [None, None]
    pend_s = [None, None]

    def issue_gather(i, b):
        src, _, g = items[i]
        pend_g[b] = pltpu.async_copy(src.at[z_v.at[g]], rows_v.at[b], gsems[b])

    issue_gather(0, 0)
    for i in range(n):
        b = i % 2
        if i + 1 < n:
            b2 = (i + 1) % 2
            if pend_s[b2] is not None:
                pend_s[b2].wait()
                pend_s[b2] = None
            issue_gather(i + 1, b2)
        pend_g[b].wait()
        _, dst, g = items[i]
        pend_s[b] = pltpu.async_copy(
            rows_v.at[b], dst.at[pl.ds(base + g * _CHUNK, _CHUNK)], ssems[b])
    for b in range(2):
        if pend_s[b] is not None:
            pend_s[b].wait()


def _sc_gather(mus_t, sig_t, z, z_off, bh):
    N, D = mus_t.shape
    out_t = jax.ShapeDtypeStruct((bh, D), jnp.float32)
    mesh = plsc.VectorSubcoreMesh(core_axis_name="c", subcore_axis_name="s")
    b_per_w = bh // _NW
    n_g = b_per_w // _CHUNK
    k = functools.partial(
        pl.kernel,
        out_type=(out_t, out_t),
        mesh=mesh,
        scratch_types=[
            pltpu.VMEM((n_g, _CHUNK), jnp.int32),
            pltpu.VMEM((2, _CHUNK, D), jnp.float32),
            pltpu.SemaphoreType.DMA,
            pltpu.SemaphoreType.DMA,
            pltpu.SemaphoreType.DMA,
            pltpu.SemaphoreType.DMA,
        ],
        compiler_params=pltpu.CompilerParams(needs_layout_passes=False),
    )(functools.partial(_sc_rowgather_body, z_off=z_off, bh=bh))
    return k(mus_t, sig_t, z)


def kernel(mus, sigmas, z):
    D, N = mus.shape
    B = z.shape[0]
    mus_t = mus.T
    sig_t = sigmas.T
    b1 = (5 * B // 8 // _TBLK) * _TBLK
    b2 = B - b1
    muz1, sigz1 = _sc_gather(mus_t, sig_t, z, 0, b1)
    muz2, sigz2 = _sc_gather(mus_t, sig_t, z, b1, b2)
    pm, ps = _tc_transpose_half1(muz1, sigz1, B)
    return _tc_transpose_half2(pm, ps, muz2, sigz2)
